# baseline (speedup 1.0000x reference)
import functools

import jax
import jax.numpy as jnp
import numpy as np
from jax import lax
from jax.experimental import pallas as pl
from jax.experimental.pallas import tpu as pltpu
from jax.experimental.pallas import tpu_sc as plsc

N_TIME = 128
N_FREQ = 32
D_MODEL = 128
N_ERP = 128
N_TFR = N_TIME * N_FREQ
N_TOKENS = N_ERP + N_TFR
BATCH = 16

_NUM_WORKERS = 32
_ROWS_PER_WORKER = 136
_N_PAD = _NUM_WORKERS * _ROWS_PER_WORKER
_CHUNKS = ((0, 128), (128, 8))


def _make_static_indices():
    t_idx = np.zeros((_N_PAD,), dtype=np.int32)
    f_idx = np.full((_N_PAD,), N_FREQ, dtype=np.int32)
    ty_idx = np.zeros((_N_PAD,), dtype=np.int32)
    t_idx[:N_ERP] = np.arange(N_ERP, dtype=np.int32)
    k = np.arange(N_TFR, dtype=np.int32)
    t_idx[N_ERP:N_TOKENS] = k // N_FREQ
    f_idx[N_ERP:N_TOKENS] = k % N_FREQ
    ty_idx[N_ERP:N_TOKENS] = 1
    return t_idx, f_idx, ty_idx


_T_IDX, _F_IDX, _TY_IDX = _make_static_indices()

_sc_mesh = plsc.VectorSubcoreMesh(core_axis_name="c", subcore_axis_name="s")


@functools.partial(
    pl.kernel,
    mesh=_sc_mesh,
    out_type=jax.ShapeDtypeStruct((_N_PAD, D_MODEL), jnp.float32),
    scratch_types=[
        pltpu.VMEM((_ROWS_PER_WORKER,), jnp.int32),
        pltpu.VMEM((_ROWS_PER_WORKER,), jnp.int32),
        pltpu.VMEM((_ROWS_PER_WORKER,), jnp.int32),
        pltpu.VMEM((_ROWS_PER_WORKER, D_MODEL), jnp.float32),
        pltpu.VMEM((_ROWS_PER_WORKER, D_MODEL), jnp.float32),
        pltpu.VMEM((_ROWS_PER_WORKER, D_MODEL), jnp.float32),
        pltpu.SemaphoreType.DMA,
    ],
)
def _pe_gather(t_idx_hbm, f_idx_hbm, ty_idx_hbm, time_hbm, freqx_hbm, type_hbm,
               pe_hbm, tix, fix, tyx, tbuf, fbuf, tybuf, sem):
    n_cores = 2
    wid = lax.axis_index("s") * n_cores + lax.axis_index("c")
    base = wid * _ROWS_PER_WORKER

    pltpu.sync_copy(t_idx_hbm.at[pl.ds(base, _ROWS_PER_WORKER)], tix)
    pltpu.sync_copy(f_idx_hbm.at[pl.ds(base, _ROWS_PER_WORKER)], fix)
    pltpu.sync_copy(ty_idx_hbm.at[pl.ds(base, _ROWS_PER_WORKER)], tyx)

    for off, ln in _CHUNKS:
        pltpu.async_copy(
            time_hbm.at[tix.at[pl.ds(off, ln)]], tbuf.at[pl.ds(off, ln)], sem
        ).wait()
        pltpu.async_copy(
            freqx_hbm.at[fix.at[pl.ds(off, ln)]], fbuf.at[pl.ds(off, ln)], sem
        ).wait()
        pltpu.async_copy(
            type_hbm.at[tyx.at[pl.ds(off, ln)]], tybuf.at[pl.ds(off, ln)], sem
        ).wait()

    def _row(r, carry):
        for j in range(D_MODEL // 16):
            s = pl.ds(j * 16, 16)
            tbuf[r, s] = tbuf[r, s] + fbuf[r, s] + tybuf[r, s]
        return carry

    lax.fori_loop(0, _ROWS_PER_WORKER, _row, 0)

    pltpu.sync_copy(tbuf, pe_hbm.at[pl.ds(base, _ROWS_PER_WORKER)])


def _add_body(x_ref, pe_ref, out_ref):
    out_ref[0] = x_ref[0] + pe_ref[...]


def _tc_add(x, pe):
    return pl.pallas_call(
        _add_body,
        grid=(BATCH,),
        in_specs=[
            pl.BlockSpec((1, N_TOKENS, D_MODEL), lambda b: (b, 0, 0)),
            pl.BlockSpec((N_TOKENS, D_MODEL), lambda b: (0, 0)),
        ],
        out_specs=pl.BlockSpec((1, N_TOKENS, D_MODEL), lambda b: (b, 0, 0)),
        out_shape=jax.ShapeDtypeStruct((BATCH, N_TOKENS, D_MODEL), jnp.float32),
    )(x, pe)


def kernel(x, time_emb, freq_emb, type_emb):
    freq_ext = jnp.concatenate(
        [freq_emb, jnp.zeros((1, D_MODEL), jnp.float32)], axis=0
    )
    pe = _pe_gather(
        jnp.asarray(_T_IDX), jnp.asarray(_F_IDX), jnp.asarray(_TY_IDX),
        time_emb, freq_ext, type_emb,
    )
    return _tc_add(x, pe)


# speedup vs baseline: 1.0193x; 1.0193x over previous
import functools

import jax
import jax.numpy as jnp
import numpy as np
from jax import lax
from jax.experimental import pallas as pl
from jax.experimental.pallas import tpu as pltpu
from jax.experimental.pallas import tpu_sc as plsc

N_TIME = 128
N_FREQ = 32
D_MODEL = 128
N_ERP = 128
N_TFR = N_TIME * N_FREQ
N_TOKENS = N_ERP + N_TFR
BATCH = 16

_NUM_WORKERS = 32
_ROWS_PER_WORKER = 136
_N_PAD = _NUM_WORKERS * _ROWS_PER_WORKER
_CHUNKS = ((0, 128), (128, 8))


def _make_static_indices():
    t_idx = np.zeros((_N_PAD,), dtype=np.int32)
    f_idx = np.full((_N_PAD,), N_FREQ, dtype=np.int32)
    ty_idx = np.zeros((_N_PAD,), dtype=np.int32)
    t_idx[:N_ERP] = np.arange(N_ERP, dtype=np.int32)
    k = np.arange(N_TFR, dtype=np.int32)
    t_idx[N_ERP:N_TOKENS] = k // N_FREQ
    f_idx[N_ERP:N_TOKENS] = k % N_FREQ
    ty_idx[N_ERP:N_TOKENS] = 1
    return t_idx, f_idx, ty_idx


_T_IDX, _F_IDX, _TY_IDX = _make_static_indices()

_sc_mesh = plsc.VectorSubcoreMesh(core_axis_name="c", subcore_axis_name="s")


@functools.partial(
    pl.kernel,
    mesh=_sc_mesh,
    out_type=jax.ShapeDtypeStruct((_N_PAD, D_MODEL), jnp.float32),
    scratch_types=[
        pltpu.VMEM((_ROWS_PER_WORKER,), jnp.int32),
        pltpu.VMEM((_ROWS_PER_WORKER,), jnp.int32),
        pltpu.VMEM((_ROWS_PER_WORKER,), jnp.int32),
        pltpu.VMEM((_ROWS_PER_WORKER, D_MODEL), jnp.float32),
        pltpu.VMEM((_ROWS_PER_WORKER, D_MODEL), jnp.float32),
        pltpu.VMEM((_ROWS_PER_WORKER, D_MODEL), jnp.float32),
        pltpu.SemaphoreType.DMA,
    ],
)
def _pe_gather(t_idx_hbm, f_idx_hbm, ty_idx_hbm, time_hbm, freqx_hbm, type_hbm,
               pe_hbm, tix, fix, tyx, tbuf, fbuf, tybuf, sem):
    n_cores = 2
    wid = lax.axis_index("s") * n_cores + lax.axis_index("c")
    base = wid * _ROWS_PER_WORKER

    pltpu.sync_copy(t_idx_hbm.at[pl.ds(base, _ROWS_PER_WORKER)], tix)
    pltpu.sync_copy(f_idx_hbm.at[pl.ds(base, _ROWS_PER_WORKER)], fix)
    pltpu.sync_copy(ty_idx_hbm.at[pl.ds(base, _ROWS_PER_WORKER)], tyx)

    handles = []
    for off, ln in _CHUNKS:
        sl = pl.ds(off, ln)
        handles.append(pltpu.async_copy(time_hbm.at[tix.at[sl]], tbuf.at[sl], sem))
        handles.append(pltpu.async_copy(freqx_hbm.at[fix.at[sl]], fbuf.at[sl], sem))
        handles.append(pltpu.async_copy(type_hbm.at[tyx.at[sl]], tybuf.at[sl], sem))
    for h in handles:
        h.wait()

    def _rows(i, carry):
        r0 = i * 4
        for k in range(4):
            for j in range(D_MODEL // 16):
                s = pl.ds(j * 16, 16)
                tbuf[r0 + k, s] = tbuf[r0 + k, s] + fbuf[r0 + k, s] + tybuf[r0 + k, s]
        return carry

    lax.fori_loop(0, _ROWS_PER_WORKER // 4, _rows, 0)

    pltpu.sync_copy(tbuf, pe_hbm.at[pl.ds(base, _ROWS_PER_WORKER)])


def _add_body(x_ref, pe_ref, out_ref):
    out_ref[0] = x_ref[0] + pe_ref[...]


def _tc_add(x, pe):
    return pl.pallas_call(
        _add_body,
        grid=(BATCH,),
        in_specs=[
            pl.BlockSpec((1, N_TOKENS, D_MODEL), lambda b: (b, 0, 0)),
            pl.BlockSpec((N_TOKENS, D_MODEL), lambda b: (0, 0)),
        ],
        out_specs=pl.BlockSpec((1, N_TOKENS, D_MODEL), lambda b: (b, 0, 0)),
        out_shape=jax.ShapeDtypeStruct((BATCH, N_TOKENS, D_MODEL), jnp.float32),
    )(x, pe)


def kernel(x, time_emb, freq_emb, type_emb):
    freq_ext = jnp.concatenate(
        [freq_emb, jnp.zeros((1, D_MODEL), jnp.float32)], axis=0
    )
    pe = _pe_gather(
        jnp.asarray(_T_IDX), jnp.asarray(_F_IDX), jnp.asarray(_TY_IDX),
        time_emb, freq_ext, type_emb,
    )
    return _tc_add(x, pe)


# speedup vs baseline: 3.8007x; 3.7287x over previous
import functools

import jax
import jax.numpy as jnp
from jax import lax
from jax.experimental import pallas as pl
from jax.experimental.pallas import tpu as pltpu
from jax.experimental.pallas import tpu_sc as plsc

N_TIME = 128
N_FREQ = 32
D_MODEL = 128
N_ERP = 128
N_TFR = N_TIME * N_FREQ
N_TOKENS = N_ERP + N_TFR
BATCH = 16

_NUM_WORKERS = 32
_ROWS_PER_WORKER = 136
_N_PAD = _NUM_WORKERS * _ROWS_PER_WORKER

_sc_mesh = plsc.VectorSubcoreMesh(core_axis_name="c", subcore_axis_name="s")


@functools.partial(
    pl.kernel,
    mesh=_sc_mesh,
    out_type=jax.ShapeDtypeStruct((_N_PAD, D_MODEL), jnp.float32),
    scratch_types=[
        pltpu.VMEM((N_TIME, D_MODEL), jnp.float32),
        pltpu.VMEM((N_FREQ + 1, D_MODEL), jnp.float32),
        pltpu.VMEM((2, D_MODEL), jnp.float32),
        pltpu.VMEM((_ROWS_PER_WORKER, D_MODEL), jnp.float32),
        pltpu.SemaphoreType.DMA,
    ],
)
def _pe_gather(time_hbm, freqx_hbm, type_hbm, pe_hbm,
               timev, freqv, typev, obuf, sem):
    n_cores = 2
    wid = lax.axis_index("s") * n_cores + lax.axis_index("c")
    base = wid * _ROWS_PER_WORKER

    handles = [
        pltpu.async_copy(time_hbm, timev, sem),
        pltpu.async_copy(freqx_hbm, freqv, sem),
        pltpu.async_copy(type_hbm, typev, sem),
    ]
    for h in handles:
        h.wait()

    def _row(r, carry):
        n = base + r
        k = n - N_ERP
        erp = n < N_ERP
        t = jnp.where(erp, n, jnp.minimum(k >> 5, N_TIME - 1))
        f = jnp.where(erp, N_FREQ, k & (N_FREQ - 1))
        ty = jnp.where(erp, 0, 1)
        for j in range(D_MODEL // 16):
            s = pl.ds(j * 16, 16)
            obuf[r, s] = timev[t, s] + freqv[f, s] + typev[ty, s]
        return carry

    lax.fori_loop(0, _ROWS_PER_WORKER, _row, 0)

    pltpu.sync_copy(obuf, pe_hbm.at[pl.ds(base, _ROWS_PER_WORKER)])


def _add_body(x_ref, pe_ref, out_ref):
    out_ref[0] = x_ref[0] + pe_ref[...]


def _tc_add(x, pe):
    return pl.pallas_call(
        _add_body,
        grid=(BATCH,),
        in_specs=[
            pl.BlockSpec((1, N_TOKENS, D_MODEL), lambda b: (b, 0, 0)),
            pl.BlockSpec((N_TOKENS, D_MODEL), lambda b: (0, 0)),
        ],
        out_specs=pl.BlockSpec((1, N_TOKENS, D_MODEL), lambda b: (b, 0, 0)),
        out_shape=jax.ShapeDtypeStruct((BATCH, N_TOKENS, D_MODEL), jnp.float32),
    )(x, pe)


def kernel(x, time_emb, freq_emb, type_emb):
    freq_ext = jnp.concatenate(
        [freq_emb, jnp.zeros((1, D_MODEL), jnp.float32)], axis=0
    )
    pe = _pe_gather(time_emb, freq_ext, type_emb)
    return _tc_add(x, pe)


# speedup vs baseline: 6.1462x; 1.6171x over previous
import jax
import jax.numpy as jnp
from jax.experimental import pallas as pl

N_TIME = 128
N_FREQ = 32
D_MODEL = 128
N_ERP = 128
N_TFR = N_TIME * N_FREQ
N_TOKENS = N_ERP + N_TFR
BATCH = 16

_HALF = N_TOKENS // 2
_T0 = (_HALF - N_ERP) // N_FREQ


def _body(x_ref, time_ref, freq_ref, type_ref, out_ref):
    j = pl.program_id(1)
    t = time_ref[...]
    ty0 = type_ref[0:1, :]
    ty1 = type_ref[1:2, :]
    f = freq_ref[...]

    @pl.when(j == 0)
    def _first_half():
        out_ref[0, :N_ERP, :] = x_ref[0, :N_ERP, :] + (t + ty0)
        xr = x_ref[0, N_ERP:, :].reshape(_T0, N_FREQ, D_MODEL)
        pe = xr + t[:_T0][:, None, :] + f[None, :, :] + ty1[None, :, :]
        out_ref[0, N_ERP:, :] = pe.reshape(_HALF - N_ERP, D_MODEL)

    @pl.when(j == 1)
    def _second_half():
        xr = x_ref[0].reshape(N_TIME - _T0, N_FREQ, D_MODEL)
        pe = xr + t[_T0:][:, None, :] + f[None, :, :] + ty1[None, :, :]
        out_ref[0] = pe.reshape(_HALF, D_MODEL)


def kernel(x, time_emb, freq_emb, type_emb):
    return pl.pallas_call(
        _body,
        grid=(BATCH, 2),
        in_specs=[
            pl.BlockSpec((1, _HALF, D_MODEL), lambda b, j: (b, j, 0)),
            pl.BlockSpec((N_TIME, D_MODEL), lambda b, j: (0, 0)),
            pl.BlockSpec((N_FREQ, D_MODEL), lambda b, j: (0, 0)),
            pl.BlockSpec((2, D_MODEL), lambda b, j: (0, 0)),
        ],
        out_specs=pl.BlockSpec((1, _HALF, D_MODEL), lambda b, j: (b, j, 0)),
        out_shape=jax.ShapeDtypeStruct((BATCH, N_TOKENS, D_MODEL), jnp.float32),
    )(x, time_emb, freq_emb, type_emb)


# speedup vs baseline: 9.1847x; 1.4944x over previous
import jax
import jax.numpy as jnp
from jax.experimental import pallas as pl

N_TIME = 128
N_FREQ = 32
D_MODEL = 128
N_ERP = 128
N_TFR = N_TIME * N_FREQ
N_TOKENS = N_ERP + N_TFR
BATCH = 16


def _body(x_ref, time_ref, freq_ref, type_ref, out_ref):
    t = time_ref[...]
    ty0 = type_ref[0:1, :]
    ty1 = type_ref[1:2, :]
    f = freq_ref[...]

    for b in range(2):
        out_ref[b, :N_ERP, :] = x_ref[b, :N_ERP, :] + (t + ty0)
        xr = x_ref[b, N_ERP:, :].reshape(N_TIME, N_FREQ, D_MODEL)
        pe_tfr = xr + t[:, None, :] + f[None, :, :] + ty1[None, :, :]
        out_ref[b, N_ERP:, :] = pe_tfr.reshape(N_TFR, D_MODEL)


def kernel(x, time_emb, freq_emb, type_emb):
    return pl.pallas_call(
        _body,
        grid=(BATCH // 2,),
        in_specs=[
            pl.BlockSpec((2, N_TOKENS, D_MODEL), lambda b: (b, 0, 0)),
            pl.BlockSpec((N_TIME, D_MODEL), lambda b: (0, 0)),
            pl.BlockSpec((N_FREQ, D_MODEL), lambda b: (0, 0)),
            pl.BlockSpec((2, D_MODEL), lambda b: (0, 0)),
        ],
        out_specs=pl.BlockSpec((2, N_TOKENS, D_MODEL), lambda b: (b, 0, 0)),
        out_shape=jax.ShapeDtypeStruct((BATCH, N_TOKENS, D_MODEL), jnp.float32),
    )(x, time_emb, freq_emb, type_emb)
